# Initial kernel scaffold; baseline (speedup 1.0000x reference)
import jax
import jax.numpy as jnp
from jax.experimental import pallas as pl


def kernel(x_seq):
    raise NotImplementedError("write your pallas kernel here")


# speedup vs baseline: 1.0853x; 1.0853x over previous
import jax
import jax.numpy as jnp
from jax.experimental import pallas as pl
from jax.experimental.pallas import tpu as pltpu

_T_REMAIN = 12


def _copy_kernel(sec_ref, x_ref, o_ref):
    o_ref[...] = x_ref[...]


def kernel(x_seq):
    T, N, C, H, W = x_seq.shape
    row = N * C * H * W
    x3 = x_seq.reshape(T, row // W, W)

    idx_key = jax.random.key(42)
    sec_list = jnp.sort(jax.random.choice(idx_key, T, shape=(_T_REMAIN,), replace=False))

    n_chunks = 8
    S = (row // W) // n_chunks

    grid_spec = pltpu.PrefetchScalarGridSpec(
        num_scalar_prefetch=1,
        grid=(_T_REMAIN, n_chunks),
        in_specs=[
            pl.BlockSpec((1, S, W), lambda i, j, sec: (sec[i], j, 0)),
        ],
        out_specs=pl.BlockSpec((1, S, W), lambda i, j, sec: (i, j, 0)),
    )

    out = pl.pallas_call(
        _copy_kernel,
        grid_spec=grid_spec,
        out_shape=jax.ShapeDtypeStruct((_T_REMAIN, row // W, W), x_seq.dtype),
    )(sec_list, x3)

    return out.reshape(_T_REMAIN, N, C, H, W)
